# baseline (device time: 92937 ns/iter reference)
import jax
import jax.numpy as jnp
from jax import lax
from jax.experimental import pallas as pl
from jax.experimental.pallas import tpu as pltpu

N_DEV = 16
M = 512
CHUNK = M // N_DEV


def kernel(dy, W):
    def body(dy_ref, w_ref, out_ref, send_buf, rs_bufs, ag_bufs,
             rs_send_sems, rs_recv_sems, ag_send_sems, ag_recv_sems):
        my = lax.axis_index("i")
        right = lax.rem(my + 1, N_DEV)

        a = dy_ref[...].astype(jnp.bfloat16)
        b = w_ref[...].astype(jnp.bfloat16)
        out_ref[...] = lax.dot_general(
            a, b, (((1,), (1,)), ((), ())),
            preferred_element_type=jnp.float32,
        )

        for s in range(N_DEV - 1):
            send_chunk = lax.rem(my - s + 2 * N_DEV, N_DEV)
            recv_chunk = lax.rem(my - s - 1 + 2 * N_DEV, N_DEV)
            send_buf[...] = out_ref[pl.ds(send_chunk * CHUNK, CHUNK), :]
            rdma = pltpu.make_async_remote_copy(
                src_ref=send_buf,
                dst_ref=rs_bufs.at[s],
                send_sem=rs_send_sems.at[s],
                recv_sem=rs_recv_sems.at[s],
                device_id=(right,),
                device_id_type=pl.DeviceIdType.MESH,
            )
            rdma.start()
            rdma.wait()
            out_ref[pl.ds(recv_chunk * CHUNK, CHUNK), :] = (
                out_ref[pl.ds(recv_chunk * CHUNK, CHUNK), :] + rs_bufs[s]
            )

        for s in range(N_DEV - 1):
            send_chunk = lax.rem(my + 1 - s + 2 * N_DEV, N_DEV)
            recv_chunk = lax.rem(my - s + 2 * N_DEV, N_DEV)
            send_buf[...] = out_ref[pl.ds(send_chunk * CHUNK, CHUNK), :]
            rdma = pltpu.make_async_remote_copy(
                src_ref=send_buf,
                dst_ref=ag_bufs.at[s],
                send_sem=ag_send_sems.at[s],
                recv_sem=ag_recv_sems.at[s],
                device_id=(right,),
                device_id_type=pl.DeviceIdType.MESH,
            )
            rdma.start()
            rdma.wait()
            out_ref[pl.ds(recv_chunk * CHUNK, CHUNK), :] = ag_bufs[s]

    return pl.pallas_call(
        body,
        out_shape=jax.ShapeDtypeStruct((M, M), jnp.float32),
        in_specs=[
            pl.BlockSpec(memory_space=pltpu.VMEM),
            pl.BlockSpec(memory_space=pltpu.VMEM),
        ],
        out_specs=pl.BlockSpec(memory_space=pltpu.VMEM),
        scratch_shapes=[
            pltpu.VMEM((CHUNK, M), jnp.float32),
            pltpu.VMEM((N_DEV - 1, CHUNK, M), jnp.float32),
            pltpu.VMEM((N_DEV - 1, CHUNK, M), jnp.float32),
            pltpu.SemaphoreType.DMA((N_DEV - 1,)),
            pltpu.SemaphoreType.DMA((N_DEV - 1,)),
            pltpu.SemaphoreType.DMA((N_DEV - 1,)),
            pltpu.SemaphoreType.DMA((N_DEV - 1,)),
        ],
    )(dy, W)


# device time: 38780 ns/iter; 2.3965x vs baseline; 2.3965x over previous
import jax
import jax.numpy as jnp
from jax import lax
from jax.experimental import pallas as pl
from jax.experimental.pallas import tpu as pltpu

N_DEV = 16
M = 512
CHUNK = M // N_DEV


def kernel(dy, W):
    def body(dy_ref, w_ref, out_ref, rs_bufs,
             rs_send_sems, rs_recv_sems, ag_send_sems, ag_recv_sems):
        my = lax.axis_index("i")

        a = dy_ref[...].astype(jnp.bfloat16)
        b = w_ref[...].astype(jnp.bfloat16)
        out_ref[...] = lax.dot_general(
            a, b, (((1,), (1,)), ((), ())),
            preferred_element_type=jnp.float32,
        )

        rs = []
        for k in range(1, N_DEV):
            tgt = lax.rem(my + k, N_DEV)
            rdma = pltpu.make_async_remote_copy(
                src_ref=out_ref.at[pl.ds(tgt * CHUNK, CHUNK), :],
                dst_ref=rs_bufs.at[N_DEV - 1 - k],
                send_sem=rs_send_sems.at[k - 1],
                recv_sem=rs_recv_sems.at[N_DEV - 1 - k],
                device_id=(tgt,),
                device_id_type=pl.DeviceIdType.MESH,
            )
            rdma.start()
            rs.append(rdma)
        for rdma in rs:
            rdma.wait()

        acc = out_ref[pl.ds(my * CHUNK, CHUNK), :]
        for k in range(N_DEV - 1):
            acc = acc + rs_bufs[k]
        out_ref[pl.ds(my * CHUNK, CHUNK), :] = acc

        ag = []
        for k in range(1, N_DEV):
            tgt = lax.rem(my + k, N_DEV)
            rdma = pltpu.make_async_remote_copy(
                src_ref=out_ref.at[pl.ds(my * CHUNK, CHUNK), :],
                dst_ref=out_ref.at[pl.ds(my * CHUNK, CHUNK), :],
                send_sem=ag_send_sems.at[k - 1],
                recv_sem=ag_recv_sems.at[N_DEV - 1 - k],
                device_id=(tgt,),
                device_id_type=pl.DeviceIdType.MESH,
            )
            rdma.start()
            ag.append(rdma)
        for rdma in ag:
            rdma.wait()

    return pl.pallas_call(
        body,
        out_shape=jax.ShapeDtypeStruct((M, M), jnp.float32),
        in_specs=[
            pl.BlockSpec(memory_space=pltpu.VMEM),
            pl.BlockSpec(memory_space=pltpu.VMEM),
        ],
        out_specs=pl.BlockSpec(memory_space=pltpu.VMEM),
        scratch_shapes=[
            pltpu.VMEM((N_DEV - 1, CHUNK, M), jnp.float32),
            pltpu.SemaphoreType.DMA((N_DEV - 1,)),
            pltpu.SemaphoreType.DMA((N_DEV - 1,)),
            pltpu.SemaphoreType.DMA((N_DEV - 1,)),
            pltpu.SemaphoreType.DMA((N_DEV - 1,)),
        ],
    )(dy, W)


# device time: 29989 ns/iter; 3.0990x vs baseline; 1.2931x over previous
import jax
import jax.numpy as jnp
from jax import lax
from jax.experimental import pallas as pl
from jax.experimental.pallas import tpu as pltpu

N_DEV = 16
M = 512
CHUNK = M // N_DEV


def kernel(dy, W):
    def body(dy_ref, w_ref, out_ref, pbf, rs_bufs, ag_send, ag_bufs,
             rs_send_sems, rs_recv_sems, ag_send_sems, ag_recv_sems):
        my = lax.axis_index("i")

        a = dy_ref[...].astype(jnp.bfloat16)
        b = w_ref[...].astype(jnp.bfloat16)
        partial = lax.dot_general(
            a, b, (((1,), (1,)), ((), ())),
            preferred_element_type=jnp.float32,
        )
        out_ref[...] = partial
        pbf[...] = partial.astype(jnp.bfloat16)

        rs = []
        for k in range(1, N_DEV):
            tgt = lax.rem(my + k, N_DEV)
            rdma = pltpu.make_async_remote_copy(
                src_ref=pbf.at[pl.ds(tgt * CHUNK, CHUNK), :],
                dst_ref=rs_bufs.at[N_DEV - 1 - k],
                send_sem=rs_send_sems.at[k - 1],
                recv_sem=rs_recv_sems.at[N_DEV - 1 - k],
                device_id=(tgt,),
                device_id_type=pl.DeviceIdType.MESH,
            )
            rdma.start()
            rs.append(rdma)
        for rdma in rs:
            rdma.wait()

        acc = out_ref[pl.ds(my * CHUNK, CHUNK), :]
        for k in range(N_DEV - 1):
            acc = acc + rs_bufs[k].astype(jnp.float32)
        out_ref[pl.ds(my * CHUNK, CHUNK), :] = acc
        ag_send[...] = acc.astype(jnp.bfloat16)

        ag = []
        for k in range(1, N_DEV):
            tgt = lax.rem(my + k, N_DEV)
            rdma = pltpu.make_async_remote_copy(
                src_ref=ag_send,
                dst_ref=ag_bufs.at[N_DEV - 1 - k],
                send_sem=ag_send_sems.at[k - 1],
                recv_sem=ag_recv_sems.at[N_DEV - 1 - k],
                device_id=(tgt,),
                device_id_type=pl.DeviceIdType.MESH,
            )
            rdma.start()
            ag.append(rdma)
        for k, rdma in enumerate(ag, start=1):
            rdma.wait()
            j = N_DEV - 1 - k
            src = lax.rem(my - k + N_DEV, N_DEV)
            out_ref[pl.ds(src * CHUNK, CHUNK), :] = (
                ag_bufs[j].astype(jnp.float32)
            )

    return pl.pallas_call(
        body,
        out_shape=jax.ShapeDtypeStruct((M, M), jnp.float32),
        in_specs=[
            pl.BlockSpec(memory_space=pltpu.VMEM),
            pl.BlockSpec(memory_space=pltpu.VMEM),
        ],
        out_specs=pl.BlockSpec(memory_space=pltpu.VMEM),
        scratch_shapes=[
            pltpu.VMEM((M, M), jnp.bfloat16),
            pltpu.VMEM((N_DEV - 1, CHUNK, M), jnp.bfloat16),
            pltpu.VMEM((CHUNK, M), jnp.bfloat16),
            pltpu.VMEM((N_DEV - 1, CHUNK, M), jnp.bfloat16),
            pltpu.SemaphoreType.DMA((N_DEV - 1,)),
            pltpu.SemaphoreType.DMA((N_DEV - 1,)),
            pltpu.SemaphoreType.DMA((N_DEV - 1,)),
            pltpu.SemaphoreType.DMA((N_DEV - 1,)),
        ],
    )(dy, W)


# device time: 24644 ns/iter; 3.7712x vs baseline; 1.2169x over previous
import jax
import jax.numpy as jnp
from jax import lax
from jax.experimental import pallas as pl
from jax.experimental.pallas import tpu as pltpu

N_DEV = 16
M = 512
CHUNK = M // N_DEV


def kernel(dy, W):
    def body(dy_ref, w_ref, out_ref, pbf, rs_bufs, ag_send, ag_bufs,
             rs_send_sems, rs_recv_sems, ag_send_sems, ag_recv_sems):
        my = lax.axis_index("i")

        barrier_sem = pltpu.get_barrier_semaphore()
        for k in range(1, N_DEV):
            pl.semaphore_signal(
                barrier_sem, inc=1,
                device_id=(lax.rem(my + k, N_DEV),),
                device_id_type=pl.DeviceIdType.MESH,
            )

        a = dy_ref[...].astype(jnp.bfloat16)
        b = w_ref[...].astype(jnp.bfloat16)
        partial = lax.dot_general(
            a, b, (((1,), (1,)), ((), ())),
            preferred_element_type=jnp.float32,
        )
        out_ref[...] = partial
        pbf[...] = partial.astype(jnp.bfloat16)

        pl.semaphore_wait(barrier_sem, N_DEV - 1)

        rs = []
        for k in range(1, N_DEV):
            tgt = lax.rem(my + k, N_DEV)
            rdma = pltpu.make_async_remote_copy(
                src_ref=pbf.at[pl.ds(tgt * CHUNK, CHUNK), :],
                dst_ref=rs_bufs.at[N_DEV - 1 - k],
                send_sem=rs_send_sems.at[k - 1],
                recv_sem=rs_recv_sems.at[N_DEV - 1 - k],
                device_id=(tgt,),
                device_id_type=pl.DeviceIdType.MESH,
            )
            rdma.start()
            rs.append(rdma)

        acc = out_ref[pl.ds(my * CHUNK, CHUNK), :]
        for k, rdma in enumerate(rs, start=1):
            rdma.wait_recv()
            acc = acc + rs_bufs[N_DEV - 1 - k].astype(jnp.float32)
        out_ref[pl.ds(my * CHUNK, CHUNK), :] = acc
        ag_send[...] = acc.astype(jnp.bfloat16)

        ag = []
        for k in range(1, N_DEV):
            tgt = lax.rem(my + k, N_DEV)
            rdma = pltpu.make_async_remote_copy(
                src_ref=ag_send,
                dst_ref=ag_bufs.at[N_DEV - 1 - k],
                send_sem=ag_send_sems.at[k - 1],
                recv_sem=ag_recv_sems.at[N_DEV - 1 - k],
                device_id=(tgt,),
                device_id_type=pl.DeviceIdType.MESH,
            )
            rdma.start()
            ag.append(rdma)
        for k, rdma in enumerate(ag, start=1):
            rdma.wait_recv()
            j = N_DEV - 1 - k
            src = lax.rem(my - k + N_DEV, N_DEV)
            out_ref[pl.ds(src * CHUNK, CHUNK), :] = (
                ag_bufs[j].astype(jnp.float32)
            )

        for rdma in rs:
            rdma.wait_send()
        for rdma in ag:
            rdma.wait_send()

    return pl.pallas_call(
        body,
        out_shape=jax.ShapeDtypeStruct((M, M), jnp.float32),
        in_specs=[
            pl.BlockSpec(memory_space=pltpu.VMEM),
            pl.BlockSpec(memory_space=pltpu.VMEM),
        ],
        out_specs=pl.BlockSpec(memory_space=pltpu.VMEM),
        scratch_shapes=[
            pltpu.VMEM((M, M), jnp.bfloat16),
            pltpu.VMEM((N_DEV - 1, CHUNK, M), jnp.bfloat16),
            pltpu.VMEM((CHUNK, M), jnp.bfloat16),
            pltpu.VMEM((N_DEV - 1, CHUNK, M), jnp.bfloat16),
            pltpu.SemaphoreType.DMA((N_DEV - 1,)),
            pltpu.SemaphoreType.DMA((N_DEV - 1,)),
            pltpu.SemaphoreType.DMA((N_DEV - 1,)),
            pltpu.SemaphoreType.DMA((N_DEV - 1,)),
        ],
        compiler_params=pltpu.CompilerParams(collective_id=0),
    )(dy, W)


# device time: 23746 ns/iter; 3.9138x vs baseline; 1.0378x over previous
import jax
import jax.numpy as jnp
from jax import lax
from jax.experimental import pallas as pl
from jax.experimental.pallas import tpu as pltpu

N_DEV = 16
M = 512
CHUNK = M // N_DEV


def kernel(dy, W):
    def body(dy_ref, w_ref, out_ref, pbf, rs_bufs, ag_send, ag_bufs,
             rs_send_sems, rs_recv_sems, ag_send_sems, ag_recv_sems,
             entry_sems):
        my = lax.axis_index("i")

        barrier_sem = pltpu.get_barrier_semaphore()
        pl.semaphore_signal(barrier_sem, inc=1)
        pl.semaphore_wait(barrier_sem, 1)

        for k in range(1, N_DEV):
            pl.semaphore_signal(
                entry_sems.at[N_DEV - 1 - k], inc=1,
                device_id=(lax.rem(my + k, N_DEV),),
                device_id_type=pl.DeviceIdType.MESH,
            )

        a = dy_ref[...].astype(jnp.bfloat16)
        b = w_ref[...].astype(jnp.bfloat16)
        partial = lax.dot_general(
            a, b, (((1,), (1,)), ((), ())),
            preferred_element_type=jnp.float32,
        )
        out_ref[...] = partial
        pbf[...] = partial.astype(jnp.bfloat16)

        rs = []
        for k in range(1, N_DEV):
            tgt = lax.rem(my + k, N_DEV)
            pl.semaphore_wait(entry_sems.at[k - 1], 1)
            rdma = pltpu.make_async_remote_copy(
                src_ref=pbf.at[pl.ds(tgt * CHUNK, CHUNK), :],
                dst_ref=rs_bufs.at[N_DEV - 1 - k],
                send_sem=rs_send_sems.at[k - 1],
                recv_sem=rs_recv_sems.at[N_DEV - 1 - k],
                device_id=(tgt,),
                device_id_type=pl.DeviceIdType.MESH,
            )
            rdma.start()
            rs.append(rdma)

        acc = out_ref[pl.ds(my * CHUNK, CHUNK), :]
        for k, rdma in enumerate(rs, start=1):
            rdma.wait_recv()
            acc = acc + rs_bufs[N_DEV - 1 - k].astype(jnp.float32)
        out_ref[pl.ds(my * CHUNK, CHUNK), :] = acc
        ag_send[...] = acc.astype(jnp.bfloat16)

        ag = []
        for k in range(1, N_DEV):
            tgt = lax.rem(my + k, N_DEV)
            rdma = pltpu.make_async_remote_copy(
                src_ref=ag_send,
                dst_ref=ag_bufs.at[N_DEV - 1 - k],
                send_sem=ag_send_sems.at[k - 1],
                recv_sem=ag_recv_sems.at[N_DEV - 1 - k],
                device_id=(tgt,),
                device_id_type=pl.DeviceIdType.MESH,
            )
            rdma.start()
            ag.append(rdma)
        for k, rdma in enumerate(ag, start=1):
            rdma.wait_recv()
            j = N_DEV - 1 - k
            src = lax.rem(my - k + N_DEV, N_DEV)
            out_ref[pl.ds(src * CHUNK, CHUNK), :] = (
                ag_bufs[j].astype(jnp.float32)
            )

        for rdma in rs:
            rdma.wait_send()
        for rdma in ag:
            rdma.wait_send()

    return pl.pallas_call(
        body,
        out_shape=jax.ShapeDtypeStruct((M, M), jnp.float32),
        in_specs=[
            pl.BlockSpec(memory_space=pltpu.VMEM),
            pl.BlockSpec(memory_space=pltpu.VMEM),
        ],
        out_specs=pl.BlockSpec(memory_space=pltpu.VMEM),
        scratch_shapes=[
            pltpu.VMEM((M, M), jnp.bfloat16),
            pltpu.VMEM((N_DEV - 1, CHUNK, M), jnp.bfloat16),
            pltpu.VMEM((CHUNK, M), jnp.bfloat16),
            pltpu.VMEM((N_DEV - 1, CHUNK, M), jnp.bfloat16),
            pltpu.SemaphoreType.DMA((N_DEV - 1,)),
            pltpu.SemaphoreType.DMA((N_DEV - 1,)),
            pltpu.SemaphoreType.DMA((N_DEV - 1,)),
            pltpu.SemaphoreType.DMA((N_DEV - 1,)),
            pltpu.SemaphoreType.REGULAR((N_DEV - 1,)),
        ],
        compiler_params=pltpu.CompilerParams(collective_id=0),
    )(dy, W)


# device time: 23673 ns/iter; 3.9259x vs baseline; 1.0031x over previous
import jax
import jax.numpy as jnp
from jax import lax
from jax.experimental import pallas as pl
from jax.experimental.pallas import tpu as pltpu

N_DEV = 16
M = 512
CHUNK = M // N_DEV


def kernel(dy, W):
    def body(dy_ref, w_ref, out_ref, pbf, rs_bufs, ag_send, ag_bufs,
             rs_send_sems, rs_recv_sems, ag_send_sems, ag_recv_sems,
             entry_sems):
        my = lax.axis_index("i")

        barrier_sem = pltpu.get_barrier_semaphore()
        pl.semaphore_signal(barrier_sem, inc=1)
        pl.semaphore_wait(barrier_sem, 1)

        for k in range(1, N_DEV):
            pl.semaphore_signal(
                entry_sems.at[N_DEV - 1 - k], inc=1,
                device_id=(lax.rem(my + k, N_DEV),),
                device_id_type=pl.DeviceIdType.MESH,
            )

        a = dy_ref[...].astype(jnp.bfloat16)
        b = w_ref[...].astype(jnp.bfloat16)
        partial = lax.dot_general(
            a, b, (((1,), (1,)), ((), ())),
            preferred_element_type=jnp.float32,
        )
        pbf[...] = partial.astype(jnp.bfloat16)

        rs = []
        for k in range(1, N_DEV):
            tgt = lax.rem(my + k, N_DEV)
            pl.semaphore_wait(entry_sems.at[k - 1], 1)
            rdma = pltpu.make_async_remote_copy(
                src_ref=pbf.at[pl.ds(tgt * CHUNK, CHUNK), :],
                dst_ref=rs_bufs.at[N_DEV - 1 - k],
                send_sem=rs_send_sems.at[k - 1],
                recv_sem=rs_recv_sems.at[N_DEV - 1 - k],
                device_id=(tgt,),
                device_id_type=pl.DeviceIdType.MESH,
            )
            rdma.start()
            rs.append(rdma)

        acc = pbf[pl.ds(my * CHUNK, CHUNK), :].astype(jnp.float32)
        for k, rdma in enumerate(rs, start=1):
            rdma.wait_recv()
            acc = acc + rs_bufs[N_DEV - 1 - k].astype(jnp.float32)
        ag_send[...] = acc.astype(jnp.bfloat16)

        ag = []
        for k in range(1, N_DEV):
            tgt = lax.rem(my + k, N_DEV)
            rdma = pltpu.make_async_remote_copy(
                src_ref=ag_send,
                dst_ref=ag_bufs.at[N_DEV - 1 - k],
                send_sem=ag_send_sems.at[k - 1],
                recv_sem=ag_recv_sems.at[N_DEV - 1 - k],
                device_id=(tgt,),
                device_id_type=pl.DeviceIdType.MESH,
            )
            rdma.start()
            ag.append(rdma)
        for k, rdma in enumerate(ag, start=1):
            rdma.wait_recv()
            j = N_DEV - 1 - k
            src = lax.rem(my - k + N_DEV, N_DEV)
            out_ref[pl.ds(src * CHUNK, CHUNK), :] = (
                ag_bufs[j].astype(jnp.float32)
            )

        out_ref[pl.ds(my * CHUNK, CHUNK), :] = acc

        for rdma in rs:
            rdma.wait_send()
        for rdma in ag:
            rdma.wait_send()

    return pl.pallas_call(
        body,
        out_shape=jax.ShapeDtypeStruct((M, M), jnp.float32),
        in_specs=[
            pl.BlockSpec(memory_space=pltpu.VMEM),
            pl.BlockSpec(memory_space=pltpu.VMEM),
        ],
        out_specs=pl.BlockSpec(memory_space=pltpu.VMEM),
        scratch_shapes=[
            pltpu.VMEM((M, M), jnp.bfloat16),
            pltpu.VMEM((N_DEV - 1, CHUNK, M), jnp.bfloat16),
            pltpu.VMEM((CHUNK, M), jnp.bfloat16),
            pltpu.VMEM((N_DEV - 1, CHUNK, M), jnp.bfloat16),
            pltpu.SemaphoreType.DMA((N_DEV - 1,)),
            pltpu.SemaphoreType.DMA((N_DEV - 1,)),
            pltpu.SemaphoreType.DMA((N_DEV - 1,)),
            pltpu.SemaphoreType.DMA((N_DEV - 1,)),
            pltpu.SemaphoreType.REGULAR((N_DEV - 1,)),
        ],
        compiler_params=pltpu.CompilerParams(collective_id=0),
    )(dy, W)
